# baseline (speedup 1.0000x reference)
import functools

import jax
import jax.numpy as jnp
from jax import lax
from jax.experimental import pallas as pl
from jax.experimental.pallas import tpu as pltpu
from jax.experimental.pallas import tpu_sc as plsc

NUM_EMB = 1000000
DIM = 64
BATCH = 16384

_TILE_COLS = 128 * ((NUM_EMB + 127) // 128)
_COLS_PER_W = _TILE_COLS // 32


@functools.cache
def _build():
    info = plsc.get_sparse_core_info()
    NC, NS = info.num_cores, info.num_subcores
    NW = NC * NS
    bpw = BATCH // NW

    CPW = 31360
    PIECE = 256
    NP = CPW // PIECE
    NSUP = 31
    DUMP = BATCH
    OUT_ROWS = BATCH + 32

    mesh = plsc.VectorSubcoreMesh(core_axis_name="c", subcore_axis_name="s")

    @functools.partial(
        pl.kernel,
        mesh=mesh,
        compiler_params=pltpu.CompilerParams(needs_layout_passes=False),
        out_type=jax.ShapeDtypeStruct((OUT_ROWS, 128), jnp.float32),
        scratch_types=[
            pltpu.VMEM((4096,), jnp.int32),
            pltpu.VMEM((BATCH + 16,), jnp.int32),
            pltpu.VMEM((BATCH + 16,), jnp.int32),
            pltpu.VMEM((48,), jnp.int32),
            pltpu.VMEM((2, DIM, PIECE), jnp.float32),
            pltpu.VMEM((DIM, 128), jnp.float32),
            pltpu.VMEM((4, 16), jnp.int32),
            pltpu.SMEM((40,), jnp.int32),
            pltpu.SemaphoreType.DMA,
            pltpu.SemaphoreType.DMA,
        ],
    )
    def gather_stream(
        tt, idx_hbm, out, idxc, l1, l2, cntv, dbuf, rowbuf, sidx, smso,
        sem_f, sem_s,
    ):
        wid = lax.axis_index("s") * NC + lax.axis_index("c")
        lo = wid * CPW
        lanes = lax.iota(jnp.int32, 16)
        is_last = wid == NW - 1
        full_cnt = jnp.where(is_last, 108, NP)

        off = 0
        for big in range(BATCH // 4096):
            pltpu.sync_copy(idx_hbm.at[pl.ds(big * 4096, 4096)], idxc)

            def scan(k, off, big=big):
                c = idxc[pl.ds(k * 16, 16)]
                crel = c - lo
                m = (crel >= 0) & (crel < CPW)
                b16 = big * 4096 + k * 16 + lanes
                packed = (crel << 14) | b16
                plsc.store_compressed(l1.at[pl.ds(off, 16)], packed, mask=m)
                return off + jnp.sum(m.astype(jnp.int32))

            off = lax.fori_loop(0, 4096 // 16, scan, off)
        cnt = off

        nch = (cnt + 15) >> 4
        zero16 = jnp.zeros((16,), jnp.int32)
        for z in range(3):
            cntv[pl.ds(z * 16, 16)] = zero16

        def count(t, carry):
            e = l1[pl.ds(t * 16, 16)]
            valid = (t * 16 + lanes) < cnt
            s = e >> 24
            for i in range(NSUP):
                mi = valid & (s == i)
                pc = plsc.all_reduce_population_count(mi)
                plsc.addupdate_scatter(
                    cntv, [jnp.full((16,), i, jnp.int32)], pc, mask=lanes == 0
                )
            return carry

        lax.fori_loop(0, nch, count, 0)

        cvec = [cntv[pl.ds(z * 16, 16)] for z in range(2)]
        so = [0] * (NSUP + 1)
        for i in range(NSUP):
            so[i + 1] = so[i] + cvec[i // 16][i % 16]
        for i in range(NSUP + 1):
            smso[i] = so[i]

        def redist(t, woffs):
            e = l1[pl.ds(t * 16, 16)]
            valid = (t * 16 + lanes) < cnt
            s = e >> 24
            new = []
            for i in range(NSUP):
                mi = valid & (s == i)
                plsc.store_compressed(l2.at[pl.ds(woffs[i], 16)], e, mask=mi)
                pc = plsc.all_reduce_population_count(mi)
                new.append(woffs[i] + pc[0])
            return tuple(new)

        lax.fori_loop(0, nch, redist, tuple(so[:NSUP]))

        def piece_geom(q):
            is_full = q < full_cnt
            is_rem = (q == full_cnt) | (is_last & (q == 109))
            col = jnp.where(
                is_last & (q == 109),
                lo + 108 * PIECE + 128,
                lo + q * PIECE,
            )
            return is_full, is_rem, col

        def fetch_piece(q, par):
            is_full, is_rem, col = piece_geom(q)

            @pl.when(is_full)
            def _f():
                pltpu.async_copy(
                    tt.at[:, pl.ds(col, PIECE)], dbuf.at[par], sem_f
                )

            @pl.when(is_rem)
            def _g():
                pltpu.async_copy(
                    tt.at[:, pl.ds(col, 128)],
                    dbuf.at[par, :, pl.ds(0, 128)], sem_f
                )

        def wait_piece(q, par):
            is_full, is_rem, _ = piece_geom(q)

            @pl.when(is_full)
            def _f():
                pltpu.make_async_copy(
                    tt.at[:, pl.ds(0, PIECE)], dbuf.at[par], sem_f
                ).wait()

            @pl.when(is_rem)
            def _g():
                pltpu.make_async_copy(
                    tt.at[:, pl.ds(0, 128)],
                    dbuf.at[par, :, pl.ds(0, 128)], sem_f
                ).wait()

        fetch_piece(0, 0)

        def piece_body(p, par, ring):
            buf = dbuf.at[par]
            wait_piece(p, par)

            @pl.when(p + 1 <= NP)
            def _n():
                fetch_piece(p + 1, 1 - par)

            is_full, is_rem, col = piece_geom(p)
            width = jnp.where(is_full, PIECE, jnp.where(is_rem, 128, 0))
            pbase = col - lo
            s = p >> 2
            sbeg = smso[s]
            send = smso[s + 1]
            nseg = (send - sbeg + 15) >> 4

            def visit(u, ring):
                eo = sbeg + u * 16
                e = l2[pl.ds(eo, 16)]
                valid = (eo + lanes) < send
                crel = e >> 14
                cc = crel - pbase
                m = valid & (cc >= 0) & (cc < width)
                slot = ring & 3

                @pl.when(ring >= 4)
                def _w():
                    pltpu.make_async_copy(
                        rowbuf.at[pl.ds(0, 16), :],
                        out.at[sidx.at[0]], sem_s
                    ).wait()

                ccs = jnp.clip(cc, 0, PIECE - 1)
                bv = jnp.where(m, e & 16383, DUMP)
                plsc.store_scatter(
                    sidx, [jnp.full((16,), slot, jnp.int32), lanes], bv
                )
                rrows = slot * 16 + lanes
                for j in range(DIM):
                    x = plsc.load_gather(
                        buf, [jnp.full((16,), j, jnp.int32), ccs]
                    )
                    plsc.store_scatter(
                        rowbuf, [rrows, jnp.full((16,), j, jnp.int32)], x
                    )
                pltpu.async_copy(
                    rowbuf.at[pl.ds(slot * 16, 16), :],
                    out.at[sidx.at[slot]], sem_s
                )
                return ring + 1

            return lax.fori_loop(0, nseg, visit, ring)

        def piece_pair(p2, ring):
            ring = piece_body(2 * p2, 0, ring)
            return piece_body(2 * p2 + 1, 1, ring)

        ring = lax.fori_loop(0, (NP + 2) // 2, piece_pair, 0)

        def drain(d, carry):
            pltpu.make_async_copy(
                rowbuf.at[pl.ds(0, 16), :], out.at[sidx.at[0]], sem_s
            ).wait()
            return carry

        lax.fori_loop(0, jnp.minimum(ring, 4), drain, 0)

    return gather_stream


def kernel(table, batch_idx):
    f = _build()
    res = f(table.T, batch_idx.astype(jnp.int32))
    return res[:BATCH, :DIM]


# speedup vs baseline: 9.1493x; 9.1493x over previous
import functools

import jax
import jax.numpy as jnp
from jax import lax
from jax.experimental import pallas as pl
from jax.experimental.pallas import tpu as pltpu
from jax.experimental.pallas import tpu_sc as plsc

NUM_EMB = 1000000
DIM = 64
BATCH = 16384

_TILE_COLS = 128 * ((NUM_EMB + 127) // 128)
_COLS_PER_W = _TILE_COLS // 32


@functools.cache
def _build():
    info = plsc.get_sparse_core_info()
    NC, NS = info.num_cores, info.num_subcores
    NW = NC * NS
    bpw = BATCH // NW

    CPW = 31360
    PIECE = 256
    NP = CPW // PIECE
    NSUP = 31
    DUMP = BATCH
    OUT_ROWS = BATCH + 32

    mesh = plsc.VectorSubcoreMesh(core_axis_name="c", subcore_axis_name="s")

    @functools.partial(
        pl.kernel,
        mesh=mesh,
        compiler_params=pltpu.CompilerParams(needs_layout_passes=False),
        out_type=jax.ShapeDtypeStruct((OUT_ROWS, 128), jnp.float32),
        scratch_types=[
            pltpu.VMEM((4096,), jnp.int32),
            pltpu.VMEM((BATCH + 16,), jnp.int32),
            pltpu.VMEM((BATCH + 16,), jnp.int32),
            pltpu.VMEM((48,), jnp.int32),
            pltpu.VMEM((2, DIM, PIECE), jnp.float32),
            pltpu.VMEM((DIM, 128), jnp.float32),
            pltpu.VMEM((4, 16), jnp.int32),
            pltpu.SMEM((40,), jnp.int32),
            pltpu.SemaphoreType.DMA,
            pltpu.SemaphoreType.DMA,
        ],
    )
    def gather_stream(
        tt, idx_hbm, out, idxc, l1, l2, cntv, dbuf, rowbuf, sidx, smso,
        sem_f, sem_s,
    ):
        wid = lax.axis_index("s") * NC + lax.axis_index("c")
        lo = wid * CPW
        lanes = lax.iota(jnp.int32, 16)
        is_last = wid == NW - 1
        full_cnt = jnp.where(is_last, 108, NP)

        off = 0
        for big in range(BATCH // 4096):
            pltpu.sync_copy(idx_hbm.at[pl.ds(big * 4096, 4096)], idxc)

            def scan(k, off, big=big):
                c = idxc[pl.ds(k * 16, 16)]
                crel = c - lo
                m = (crel >= 0) & (crel < CPW)
                b16 = big * 4096 + k * 16 + lanes
                packed = (crel << 14) | b16
                plsc.store_compressed(l1.at[pl.ds(off, 16)], packed, mask=m)
                return off + jnp.sum(m.astype(jnp.int32))

            off = lax.fori_loop(0, 4096 // 16, scan, off)
        cnt = off

        nch = (cnt + 15) >> 4
        zero16 = jnp.zeros((16,), jnp.int32)
        for z in range(3):
            cntv[pl.ds(z * 16, 16)] = zero16

        def count(t, carry):
            e = l1[pl.ds(t * 16, 16)]
            valid = (t * 16 + lanes) < cnt
            s = e >> 24
            for i in range(NSUP):
                mi = valid & (s == i)
                pc = plsc.all_reduce_population_count(mi)
                plsc.addupdate_scatter(
                    cntv, [jnp.full((16,), i, jnp.int32)], pc, mask=lanes == 0
                )
            return carry

        lax.fori_loop(0, nch, count, 0)

        cvec = [cntv[pl.ds(z * 16, 16)] for z in range(2)]
        so = [0] * (NSUP + 1)
        for i in range(NSUP):
            so[i + 1] = so[i] + cvec[i // 16][i % 16]
        for i in range(NSUP + 1):
            smso[i] = so[i]

        def redist(t, woffs):
            e = l1[pl.ds(t * 16, 16)]
            valid = (t * 16 + lanes) < cnt
            s = e >> 24
            new = []
            for i in range(NSUP):
                mi = valid & (s == i)
                plsc.store_compressed(l2.at[pl.ds(woffs[i], 16)], e, mask=mi)
                pc = plsc.all_reduce_population_count(mi)
                new.append(woffs[i] + pc[0])
            return tuple(new)

        lax.fori_loop(0, nch, redist, tuple(so[:NSUP]))

        def piece_geom(q):
            is_full = q < full_cnt
            is_rem = (q == full_cnt) | (is_last & (q == 109))
            col = jnp.where(
                is_last & (q == 109),
                lo + 108 * PIECE + 128,
                lo + q * PIECE,
            )
            return is_full, is_rem, col

        def fetch_piece(q, par):
            is_full, is_rem, col = piece_geom(q)

            @pl.when(is_full)
            def _f():
                pltpu.async_copy(
                    tt.at[:, pl.ds(col, PIECE)], dbuf.at[par], sem_f
                )

            @pl.when(is_rem)
            def _g():
                pltpu.async_copy(
                    tt.at[:, pl.ds(col, 128)],
                    dbuf.at[par, :, pl.ds(0, 128)], sem_f
                )

        def wait_piece(q, par):
            is_full, is_rem, _ = piece_geom(q)

            @pl.when(is_full)
            def _f():
                pltpu.make_async_copy(
                    tt.at[:, pl.ds(0, PIECE)], dbuf.at[par], sem_f
                ).wait()

            @pl.when(is_rem)
            def _g():
                pltpu.make_async_copy(
                    tt.at[:, pl.ds(0, 128)],
                    dbuf.at[par, :, pl.ds(0, 128)], sem_f
                ).wait()

        fetch_piece(0, 0)

        def piece_body(p, par, ring):
            buf = dbuf.at[par]
            wait_piece(p, par)

            @pl.when(p + 1 <= NP)
            def _n():
                fetch_piece(p + 1, 1 - par)

            is_full, is_rem, col = piece_geom(p)
            width = jnp.where(is_full, PIECE, jnp.where(is_rem, 128, 0))
            pbase = col - lo
            s = p >> 2
            sbeg = smso[s]
            send = smso[s + 1]
            nseg = (send - sbeg + 15) >> 4

            def visit(u, ring):
                eo = sbeg + u * 16
                e = l2[pl.ds(eo, 16)]
                valid = (eo + lanes) < send
                crel = e >> 14
                cc = crel - pbase
                m = valid & (cc >= 0) & (cc < width)
                slot = ring & 3

                @pl.when(ring >= 4)
                def _w():
                    pltpu.make_async_copy(
                        rowbuf.at[pl.ds(0, 16), :],
                        out.at[sidx.at[0]], sem_s
                    ).wait()

                ccs = jnp.clip(cc, 0, PIECE - 1)
                bv = jnp.where(m, e & 16383, DUMP + ((lanes + wid) & 31))
                plsc.store_scatter(
                    sidx, [jnp.full((16,), slot, jnp.int32), lanes], bv
                )
                rrows = slot * 16 + lanes
                for j in range(DIM):
                    x = plsc.load_gather(
                        buf, [jnp.full((16,), j, jnp.int32), ccs]
                    )
                    plsc.store_scatter(
                        rowbuf, [rrows, jnp.full((16,), j, jnp.int32)], x
                    )
                pltpu.async_copy(
                    rowbuf.at[pl.ds(slot * 16, 16), :],
                    out.at[sidx.at[slot]], sem_s
                )
                return ring + 1

            return lax.fori_loop(0, nseg, visit, ring)

        def piece_pair(p2, ring):
            ring = piece_body(2 * p2, 0, ring)
            return piece_body(2 * p2 + 1, 1, ring)

        ring = lax.fori_loop(0, (NP + 2) // 2, piece_pair, 0)

        def drain(d, carry):
            pltpu.make_async_copy(
                rowbuf.at[pl.ds(0, 16), :], out.at[sidx.at[0]], sem_s
            ).wait()
            return carry

        lax.fori_loop(0, jnp.minimum(ring, 4), drain, 0)

    return gather_stream


def kernel(table, batch_idx):
    f = _build()
    res = f(table.T, batch_idx.astype(jnp.int32))
    return res[:BATCH, :DIM]
